# Initial kernel scaffold; baseline (speedup 1.0000x reference)
import functools

import jax
import jax.numpy as jnp
from jax import lax
from jax.experimental import pallas as pl
from jax.experimental.pallas import tpu as pltpu
from jax.experimental.pallas import tpu_sc as plsc

N = 10000
E = 160000
H = 256
G = 64

NC = 2
NS = 16
K = 128

ROWS_PER_TILE = N // NS

_mesh = plsc.VectorSubcoreMesh(core_axis_name="c", subcore_axis_name="s")


EPT_A = E // (NC * NS)
FULL_A = EPT_A // K
TAIL_A = EPT_A - FULL_A * K


@functools.partial(
    pl.kernel,
    out_type=jax.ShapeDtypeStruct((NC, N, 1), jnp.float32),
    mesh=_mesh,
    scratch_types=[
        pltpu.VMEM((K,), jnp.int32),
        pltpu.VMEM((TAIL_A,), jnp.int32),
        pltpu.VMEM((K, 1), jnp.float32),
        pltpu.VMEM_SHARED((N, 1), jnp.float32),
    ],
)
def _deg_kernel(cols_hbm, ones_hbm, zeros_hbm, deg_out, colv, colv_t, onesv,
                accum):
    c = lax.axis_index("c")
    s = lax.axis_index("s")
    wid = c * NS + s
    pltpu.sync_copy(ones_hbm, onesv)
    r0 = s * ROWS_PER_TILE
    pltpu.sync_copy(zeros_hbm.at[pl.ds(0, ROWS_PER_TILE)],
                    accum.at[pl.ds(r0, ROWS_PER_TILE)])
    plsc.subcore_barrier()

    base0 = wid * EPT_A

    def body(i, carry):
        base = base0 + i * K
        pltpu.sync_copy(cols_hbm.at[pl.ds(base, K)], colv)
        pltpu.sync_copy(onesv, accum.at[colv], add=True)
        return carry

    lax.fori_loop(0, FULL_A, body, 0)
    pltpu.sync_copy(cols_hbm.at[pl.ds(base0 + FULL_A * K, TAIL_A)], colv_t)
    pltpu.sync_copy(onesv.at[pl.ds(0, TAIL_A)], accum.at[colv_t], add=True)

    plsc.subcore_barrier()
    pltpu.sync_copy(accum.at[pl.ds(r0, ROWS_PER_TILE)],
                    deg_out.at[c, pl.ds(r0, ROWS_PER_TILE), :])


@functools.partial(
    pl.kernel,
    out_type=jax.ShapeDtypeStruct((NC, N, 4), jnp.float32),
    mesh=_mesh,
    scratch_types=[
        pltpu.VMEM((K,), jnp.int32),
        pltpu.VMEM((K,), jnp.int32),
        pltpu.VMEM((TAIL_A,), jnp.int32),
        pltpu.VMEM((TAIL_A,), jnp.int32),
        pltpu.VMEM((K, 4), jnp.float32),
        pltpu.VMEM_SHARED((N, 4), jnp.float32),
        pltpu.SemaphoreType.DMA,
    ],
)
def _agg1_kernel(rows_hbm, cols_hbm, q_hbm, zeros_hbm, agg_out, rowv, colv,
                 rowv_t, colv_t, gbuf, accum, sem):
    c = lax.axis_index("c")
    s = lax.axis_index("s")
    wid = c * NS + s
    r0 = s * ROWS_PER_TILE
    pltpu.sync_copy(zeros_hbm.at[pl.ds(0, ROWS_PER_TILE)],
                    accum.at[pl.ds(r0, ROWS_PER_TILE)])
    plsc.subcore_barrier()

    base0 = wid * EPT_A

    def body(i, carry):
        base = base0 + i * K
        pltpu.sync_copy(rows_hbm.at[pl.ds(base, K)], rowv)
        pltpu.sync_copy(cols_hbm.at[pl.ds(base, K)], colv)
        pltpu.async_copy(q_hbm.at[rowv], gbuf, sem).wait()
        pltpu.sync_copy(gbuf, accum.at[colv], add=True)
        return carry

    lax.fori_loop(0, FULL_A, body, 0)
    base = base0 + FULL_A * K
    pltpu.sync_copy(rows_hbm.at[pl.ds(base, TAIL_A)], rowv_t)
    pltpu.sync_copy(cols_hbm.at[pl.ds(base, TAIL_A)], colv_t)
    pltpu.async_copy(q_hbm.at[rowv_t], gbuf.at[pl.ds(0, TAIL_A)], sem).wait()
    pltpu.sync_copy(gbuf.at[pl.ds(0, TAIL_A)], accum.at[colv_t], add=True)

    plsc.subcore_barrier()
    pltpu.sync_copy(accum.at[pl.ds(r0, ROWS_PER_TILE)],
                    agg_out.at[c, pl.ds(r0, ROWS_PER_TILE), :])


HH = H // 2
EPT_C = E // NS
FULL_C = EPT_C // K
TAIL_C = EPT_C - FULL_C * K


@functools.partial(
    pl.kernel,
    out_type=jax.ShapeDtypeStruct((N, H), jnp.float32),
    mesh=_mesh,
    scratch_types=[
        pltpu.VMEM((K,), jnp.int32),
        pltpu.VMEM((K,), jnp.int32),
        pltpu.VMEM((TAIL_C,), jnp.int32),
        pltpu.VMEM((TAIL_C,), jnp.int32),
        pltpu.VMEM((K, HH), jnp.float32),
        pltpu.VMEM_SHARED((N, HH), jnp.float32),
        pltpu.SemaphoreType.DMA,
    ],
)
def _agg2_kernel(rows_hbm, cols_hbm, y0_hbm, y1_hbm, zeros_hbm, agg_out,
                 rowv, colv, rowv_t, colv_t, gbuf, accum, sem):
    c = lax.axis_index("c")
    s = lax.axis_index("s")
    r0 = s * ROWS_PER_TILE
    pltpu.sync_copy(zeros_hbm.at[pl.ds(0, ROWS_PER_TILE)],
                    accum.at[pl.ds(r0, ROWS_PER_TILE)])
    plsc.subcore_barrier()

    base0 = s * EPT_C

    def body(i, carry):
        base = base0 + i * K
        pltpu.sync_copy(rows_hbm.at[pl.ds(base, K)], rowv)
        pltpu.sync_copy(cols_hbm.at[pl.ds(base, K)], colv)

        @pl.when(c == 0)
        def _():
            pltpu.async_copy(y0_hbm.at[rowv], gbuf, sem).wait()

        @pl.when(c == 1)
        def _():
            pltpu.async_copy(y1_hbm.at[rowv], gbuf, sem).wait()

        pltpu.sync_copy(gbuf, accum.at[colv], add=True)
        return carry

    lax.fori_loop(0, FULL_C, body, 0)
    base = base0 + FULL_C * K
    pltpu.sync_copy(rows_hbm.at[pl.ds(base, TAIL_C)], rowv_t)
    pltpu.sync_copy(cols_hbm.at[pl.ds(base, TAIL_C)], colv_t)

    @pl.when(c == 0)
    def _():
        pltpu.async_copy(y0_hbm.at[rowv_t], gbuf.at[pl.ds(0, TAIL_C)],
                         sem).wait()

    @pl.when(c == 1)
    def _():
        pltpu.async_copy(y1_hbm.at[rowv_t], gbuf.at[pl.ds(0, TAIL_C)],
                         sem).wait()

    pltpu.sync_copy(gbuf.at[pl.ds(0, TAIL_C)], accum.at[colv_t], add=True)

    plsc.subcore_barrier()
    pltpu.sync_copy(accum.at[pl.ds(r0, ROWS_PER_TILE)],
                    agg_out.at[pl.ds(r0, ROWS_PER_TILE), pl.ds(c * HH, HH)])


def _prep_body(deg_ref, x_ref, wenc_ref, benc_ref, w1_ref,
               q_ref, dinv_ref, m1_ref):
    deg = deg_ref[0] + deg_ref[1] + 1.0
    dinv = lax.rsqrt(jnp.maximum(deg, 1e-12))
    dinv_ref[...] = dinv
    q_ref[...] = jnp.concatenate([x_ref[...] * dinv, dinv], axis=1)
    mst = jnp.concatenate([wenc_ref[...], benc_ref[...]], axis=0)
    m1_ref[...] = jnp.dot(mst, w1_ref[...], preferred_element_type=jnp.float32)


_prep_call = pl.pallas_call(
    _prep_body,
    out_shape=[
        jax.ShapeDtypeStruct((N, 4), jnp.float32),
        jax.ShapeDtypeStruct((N, 1), jnp.float32),
        jax.ShapeDtypeStruct((4, H), jnp.float32),
    ],
)


def _h1_body(q_ref, agg1_ref, dinv_ref, m1_ref, b1_ref, y0_ref, y1_ref):
    v = dinv_ref[...] * (agg1_ref[0] + agg1_ref[1] + q_ref[...])
    h1 = jnp.dot(v, m1_ref[...], preferred_element_type=jnp.float32)
    h1 = jnp.maximum(h1 + b1_ref[...], 0.0)
    y = dinv_ref[...] * h1
    y0_ref[...] = y[:, :HH]
    y1_ref[...] = y[:, HH:]


_h1_call = pl.pallas_call(
    _h1_body,
    out_shape=[
        jax.ShapeDtypeStruct((N, HH), jnp.float32),
        jax.ShapeDtypeStruct((N, HH), jnp.float32),
    ],
)


BLK = 1000


def _out_body(agg2_ref, y0_ref, y1_ref, dinv_ref, batch_ref, w2_ref, b2_ref,
              wc1_ref, bc1_ref, wc2_ref, bc2_ref, out_ref, pool_acc, cnt_acc):
    i = pl.program_id(0)

    @pl.when(i == 0)
    def _():
        pool_acc[...] = jnp.zeros_like(pool_acc)
        cnt_acc[...] = jnp.zeros_like(cnt_acc)

    y = jnp.concatenate([y0_ref[...], y1_ref[...]], axis=1)
    z = dinv_ref[...] * (agg2_ref[...] + y)
    h2 = jnp.dot(z, w2_ref[...], preferred_element_type=jnp.float32)
    h2 = jnp.maximum(h2 + b2_ref[...], 0.0)
    onehot = (batch_ref[...] == lax.broadcasted_iota(jnp.int32, (1, G), 1))
    onehot = onehot.astype(jnp.float32)
    pool_acc[...] += lax.dot_general(onehot, h2, (((0,), (0,)), ((), ())),
                                     preferred_element_type=jnp.float32)
    cnt_acc[...] += lax.dot_general(onehot, jnp.ones((BLK, 1), jnp.float32),
                                    (((0,), (0,)), ((), ())),
                                    preferred_element_type=jnp.float32)

    @pl.when(i == pl.num_programs(0) - 1)
    def _():
        pooled = pool_acc[...] / jnp.maximum(cnt_acc[...], 1.0)
        zc = jnp.dot(pooled, wc1_ref[...], preferred_element_type=jnp.float32)
        zc = jnp.maximum(zc + bc1_ref[...], 0.0)
        o = jnp.dot(zc, wc2_ref[...], preferred_element_type=jnp.float32)
        out_ref[...] = jax.nn.sigmoid(o + bc2_ref[...])


_out_call = pl.pallas_call(
    _out_body,
    grid=(N // BLK,),
    in_specs=[
        pl.BlockSpec((BLK, H), lambda i: (i, 0)),
        pl.BlockSpec((BLK, HH), lambda i: (i, 0)),
        pl.BlockSpec((BLK, HH), lambda i: (i, 0)),
        pl.BlockSpec((BLK, 1), lambda i: (i, 0)),
        pl.BlockSpec((BLK, 1), lambda i: (i, 0)),
        pl.BlockSpec((H, H), lambda i: (0, 0)),
        pl.BlockSpec((1, H), lambda i: (0, 0)),
        pl.BlockSpec((H, H), lambda i: (0, 0)),
        pl.BlockSpec((1, H), lambda i: (0, 0)),
        pl.BlockSpec((H, 1), lambda i: (0, 0)),
        pl.BlockSpec((1, 1), lambda i: (0, 0)),
    ],
    out_specs=pl.BlockSpec((G, 1), lambda i: (0, 0)),
    out_shape=jax.ShapeDtypeStruct((G, 1), jnp.float32),
    scratch_shapes=[
        pltpu.VMEM((G, H), jnp.float32),
        pltpu.VMEM((G, 1), jnp.float32),
    ],
)


def kernel(x, edge_index, batch, W_enc, b_enc, W1, b1, W2, b2,
           Wc1, bc1, Wc2, bc2):
    rows = edge_index[0]
    cols = edge_index[1]

    ones_k = jnp.ones((K, 1), jnp.float32)
    zeros_1 = jnp.zeros((ROWS_PER_TILE, 1), jnp.float32)
    zeros_4 = jnp.zeros((ROWS_PER_TILE, 4), jnp.float32)
    zeros_h = jnp.zeros((ROWS_PER_TILE, HH), jnp.float32)

    deg_both = _deg_kernel(cols, ones_k, zeros_1)
    q, dinv, m1aug = _prep_call(deg_both, x, W_enc,
                                b_enc.reshape(1, H), W1)
    agg1_both = _agg1_kernel(rows, cols, q, zeros_4)
    y0, y1 = _h1_call(q, agg1_both, dinv, m1aug, b1.reshape(1, H))
    agg2 = _agg2_kernel(rows, cols, y0, y1, zeros_h)
    out = _out_call(agg2, y0, y1, dinv, batch.reshape(N, 1),
                    W2, b2.reshape(1, H), Wc1, bc1.reshape(1, H),
                    Wc2, bc2.reshape(1, 1))
    return out.reshape(G)


# speedup vs baseline: 9.8167x; 9.8167x over previous
import functools

import jax
import jax.numpy as jnp
from jax import lax
from jax.experimental import pallas as pl
from jax.experimental.pallas import tpu as pltpu
from jax.experimental.pallas import tpu_sc as plsc

N = 10000
NPAD = 10240
E = 160000
H = 256
G = 64

NC = 2
NS = 16
K = 128
HH = H // 2

ROWS_PER_TILE = NPAD // NS

_mesh = plsc.VectorSubcoreMesh(core_axis_name="c", subcore_axis_name="s")


EPT_A = E // (NC * NS)
FULL_A = EPT_A // K
TAIL_A = EPT_A - FULL_A * K


@functools.partial(
    pl.kernel,
    out_type=jax.ShapeDtypeStruct((NC * NPAD,), jnp.float32),
    mesh=_mesh,
    scratch_types=[
        pltpu.VMEM((K,), jnp.int32),
        pltpu.VMEM((TAIL_A,), jnp.int32),
        pltpu.VMEM((K,), jnp.float32),
        pltpu.VMEM((ROWS_PER_TILE,), jnp.float32),
        pltpu.VMEM_SHARED((NPAD,), jnp.float32),
    ],
)
def _deg_kernel(cols_hbm, deg_out, colv, colv_t, onesv, zv, accum):
    c = lax.axis_index("c")
    s = lax.axis_index("s")
    wid = c * NS + s
    for j in range(K // 16):
        onesv[pl.ds(j * 16, 16)] = jnp.ones((16,), jnp.float32)
    for j in range(ROWS_PER_TILE // 16):
        zv[pl.ds(j * 16, 16)] = jnp.zeros((16,), jnp.float32)
    r0 = s * ROWS_PER_TILE
    pltpu.sync_copy(zv, accum.at[pl.ds(r0, ROWS_PER_TILE)])
    plsc.subcore_barrier()

    base0 = wid * EPT_A

    def body(i, carry):
        base = base0 + i * K
        pltpu.sync_copy(cols_hbm.at[pl.ds(base, K)], colv)
        pltpu.sync_copy(onesv, accum.at[colv], add=True)
        return carry

    lax.fori_loop(0, FULL_A, body, 0)
    pltpu.sync_copy(cols_hbm.at[pl.ds(base0 + FULL_A * K, TAIL_A)], colv_t)
    pltpu.sync_copy(onesv.at[pl.ds(0, TAIL_A)], accum.at[colv_t], add=True)

    plsc.subcore_barrier()
    pltpu.sync_copy(accum.at[pl.ds(r0, ROWS_PER_TILE)],
                    deg_out.at[pl.ds(c * NPAD + r0, ROWS_PER_TILE)])


EPT_C = E // NS
FULL_C = EPT_C // K
TAIL_C = EPT_C - FULL_C * K


@functools.partial(
    pl.kernel,
    out_type=jax.ShapeDtypeStruct((NPAD, H), jnp.float32),
    mesh=_mesh,
    scratch_types=[
        pltpu.VMEM((K,), jnp.int32),
        pltpu.VMEM((K,), jnp.int32),
        pltpu.VMEM((TAIL_C,), jnp.int32),
        pltpu.VMEM((TAIL_C,), jnp.int32),
        pltpu.VMEM((K, HH), jnp.float32),
        pltpu.VMEM_SHARED((NPAD, HH), jnp.float32),
        pltpu.SemaphoreType.DMA,
    ],
)
def _spmm_kernel(rows_hbm, cols_hbm, y_hbm, zeros_hbm, agg_out,
                 rowv, colv, rowv_t, colv_t, gbuf, accum, sem):
    c = lax.axis_index("c")
    s = lax.axis_index("s")
    r0 = s * ROWS_PER_TILE
    pltpu.sync_copy(zeros_hbm.at[pl.ds(0, ROWS_PER_TILE)],
                    accum.at[pl.ds(r0, ROWS_PER_TILE)])
    plsc.subcore_barrier()

    base0 = s * EPT_C
    y_half = y_hbm.at[c]

    def body(i, carry):
        base = base0 + i * K
        pltpu.sync_copy(rows_hbm.at[pl.ds(base, K)], rowv)
        pltpu.sync_copy(cols_hbm.at[pl.ds(base, K)], colv)
        pltpu.async_copy(y_half.at[rowv], gbuf, sem).wait()
        pltpu.sync_copy(gbuf, accum.at[colv], add=True)
        return carry

    lax.fori_loop(0, FULL_C, body, 0)
    base = base0 + FULL_C * K
    pltpu.sync_copy(rows_hbm.at[pl.ds(base, TAIL_C)], rowv_t)
    pltpu.sync_copy(cols_hbm.at[pl.ds(base, TAIL_C)], colv_t)
    pltpu.async_copy(y_half.at[rowv_t], gbuf.at[pl.ds(0, TAIL_C)],
                     sem).wait()
    pltpu.sync_copy(gbuf.at[pl.ds(0, TAIL_C)], accum.at[colv_t], add=True)

    plsc.subcore_barrier()
    pltpu.sync_copy(accum.at[pl.ds(r0, ROWS_PER_TILE)],
                    agg_out.at[pl.ds(r0, ROWS_PER_TILE), pl.ds(c * HH, HH)])


def _prep_body(deg_ref, dinv_ref):
    deg = deg_ref[pl.ds(0, NPAD)] + deg_ref[pl.ds(NPAD, NPAD)] + 1.0
    dinv_ref[...] = lax.rsqrt(jnp.maximum(deg, 1e-12))


_prep_call = pl.pallas_call(
    _prep_body,
    out_shape=jax.ShapeDtypeStruct((NPAD,), jnp.float32),
)

BLK = 1024
GRID = NPAD // BLK


def _pre1_body(x_ref, dinv_ref, wenc_ref, benc_ref, w1_ref, y_ref):
    h = jnp.dot(x_ref[...], wenc_ref[...], preferred_element_type=jnp.float32,
                precision=lax.Precision.HIGHEST) + benc_ref[...]
    xw = jnp.dot(h, w1_ref[...], preferred_element_type=jnp.float32,
                 precision=lax.Precision.HIGHEST)
    y = dinv_ref[...] * xw
    y_ref[0] = y[:, :HH]
    y_ref[1] = y[:, HH:]


_pre1_call = pl.pallas_call(
    _pre1_body,
    grid=(GRID,),
    in_specs=[
        pl.BlockSpec((BLK, 3), lambda i: (i, 0)),
        pl.BlockSpec((BLK, 1), lambda i: (i, 0)),
        pl.BlockSpec((3, H), lambda i: (0, 0)),
        pl.BlockSpec((1, H), lambda i: (0, 0)),
        pl.BlockSpec((H, H), lambda i: (0, 0)),
    ],
    out_specs=pl.BlockSpec((2, BLK, HH), lambda i: (0, i, 0)),
    out_shape=jax.ShapeDtypeStruct((2, NPAD, HH), jnp.float32),
)


def _mid_body(agg_ref, y_ref, dinv_ref, b1_ref, w2_ref, y2_ref):
    y1 = jnp.concatenate([y_ref[0], y_ref[1]], axis=1)
    h1 = jnp.maximum(dinv_ref[...] * (agg_ref[...] + y1) + b1_ref[...], 0.0)
    xw2 = jnp.dot(h1, w2_ref[...], preferred_element_type=jnp.float32,
                  precision=lax.Precision.HIGHEST)
    y2 = dinv_ref[...] * xw2
    y2_ref[0] = y2[:, :HH]
    y2_ref[1] = y2[:, HH:]


_mid_call = pl.pallas_call(
    _mid_body,
    grid=(GRID,),
    in_specs=[
        pl.BlockSpec((BLK, H), lambda i: (i, 0)),
        pl.BlockSpec((2, BLK, HH), lambda i: (0, i, 0)),
        pl.BlockSpec((BLK, 1), lambda i: (i, 0)),
        pl.BlockSpec((1, H), lambda i: (0, 0)),
        pl.BlockSpec((H, H), lambda i: (0, 0)),
    ],
    out_specs=pl.BlockSpec((2, BLK, HH), lambda i: (0, i, 0)),
    out_shape=jax.ShapeDtypeStruct((2, NPAD, HH), jnp.float32),
)


def _out_body(agg2_ref, y0_ref, y1_ref, dinv_ref, batch_ref, b2_ref,
              wc1_ref, bc1_ref, wc2_ref, bc2_ref, out_ref, pool_acc, cnt_acc):
    i = pl.program_id(0)

    @pl.when(i == 0)
    def _():
        pool_acc[...] = jnp.zeros_like(pool_acc)
        cnt_acc[...] = jnp.zeros_like(cnt_acc)

    y = jnp.concatenate([y0_ref[0], y1_ref[0]], axis=1)
    h2 = jnp.maximum(dinv_ref[...] * (agg2_ref[...] + y) + b2_ref[...], 0.0)
    onehot = (batch_ref[...] == lax.broadcasted_iota(jnp.int32, (1, G), 1))
    onehot = onehot.astype(jnp.float32)
    pool_acc[...] += lax.dot_general(onehot, h2, (((0,), (0,)), ((), ())),
                                     preferred_element_type=jnp.float32,
                   precision=lax.Precision.HIGHEST)
    cnt_acc[...] += lax.dot_general(onehot, jnp.ones((BLK, 1), jnp.float32),
                                    (((0,), (0,)), ((), ())),
                                    preferred_element_type=jnp.float32,
                   precision=lax.Precision.HIGHEST)

    @pl.when(i == pl.num_programs(0) - 1)
    def _():
        pooled = pool_acc[...] / jnp.maximum(cnt_acc[...], 1.0)
        zc = jnp.dot(pooled, wc1_ref[...], preferred_element_type=jnp.float32,
                   precision=lax.Precision.HIGHEST)
        zc = jnp.maximum(zc + bc1_ref[...], 0.0)
        o = jnp.dot(zc, wc2_ref[...], preferred_element_type=jnp.float32,
                   precision=lax.Precision.HIGHEST)
        out_ref[...] = jax.nn.sigmoid(o + bc2_ref[...])


_out_call = pl.pallas_call(
    _out_body,
    grid=(GRID,),
    in_specs=[
        pl.BlockSpec((BLK, H), lambda i: (i, 0)),
        pl.BlockSpec((1, BLK, HH), lambda i: (0, i, 0)),
        pl.BlockSpec((1, BLK, HH), lambda i: (1, i, 0)),
        pl.BlockSpec((BLK, 1), lambda i: (i, 0)),
        pl.BlockSpec((BLK, 1), lambda i: (i, 0)),
        pl.BlockSpec((1, H), lambda i: (0, 0)),
        pl.BlockSpec((H, H), lambda i: (0, 0)),
        pl.BlockSpec((1, H), lambda i: (0, 0)),
        pl.BlockSpec((H, 1), lambda i: (0, 0)),
        pl.BlockSpec((1, 1), lambda i: (0, 0)),
    ],
    out_specs=pl.BlockSpec((G, 1), lambda i: (0, 0)),
    out_shape=jax.ShapeDtypeStruct((G, 1), jnp.float32),
    scratch_shapes=[
        pltpu.VMEM((G, H), jnp.float32),
        pltpu.VMEM((G, 1), jnp.float32),
    ],
)


def kernel(x, edge_index, batch, W_enc, b_enc, W1, b1, W2, b2,
           Wc1, bc1, Wc2, bc2):
    rows = edge_index[0]
    cols = edge_index[1]
    xp = jnp.concatenate([x, jnp.zeros((NPAD - N, 3), x.dtype)], axis=0)
    batch_p = jnp.concatenate(
        [batch, jnp.full((NPAD - N,), G, batch.dtype)], axis=0)
    zeros_h = jnp.zeros((ROWS_PER_TILE, HH), jnp.float32)

    deg_flat = _deg_kernel(cols)
    dinv_flat = _prep_call(deg_flat)
    dinv_col = dinv_flat.reshape(NPAD, 1)
    y1 = _pre1_call(xp, dinv_col, W_enc, b_enc.reshape(1, H), W1)
    agg1 = _spmm_kernel(rows, cols, y1, zeros_h)
    y2 = _mid_call(agg1, y1, dinv_col, b1.reshape(1, H), W2)
    agg2 = _spmm_kernel(rows, cols, y2, zeros_h)
    out = _out_call(agg2, y2, y2, dinv_col, batch_p.reshape(NPAD, 1),
                    b2.reshape(1, H), Wc1, bc1.reshape(1, H),
                    Wc2, bc2.reshape(1, 1))
    return out.reshape(G)
